# baseline (speedup 1.0000x reference)
import functools

import jax
import jax.numpy as jnp
from jax import lax
from jax.experimental import pallas as pl
from jax.experimental.pallas import tpu as pltpu
from jax.experimental.pallas import tpu_sc as plsc

L = 16
NC = 2
NS = 16
NW = NC * NS


def _sc_partials(features, labels, centers):
    B, D = features.shape
    bw = B // NW
    CH = 8
    NF = 2
    NCSL = 8
    NCH = bw // CH
    KV = D // L
    RR = bw * L // 128

    mesh = plsc.VectorSubcoreMesh(core_axis_name="c", subcore_axis_name="s")

    @functools.partial(
        pl.kernel,
        mesh=mesh,
        compiler_params=pltpu.CompilerParams(use_tc_tiling_on_sc=True),
        out_type=[jax.ShapeDtypeStruct((B * L // 128, 128), jnp.float32)
                  for _ in range(3)],
        scratch_types=(
            [pltpu.VMEM((bw,), jnp.int32)]
            + [pltpu.VMEM((CH, D), jnp.float32)
               for _ in range(NF)]
            + [pltpu.VMEM((CH, D), jnp.float32)
               for _ in range(NCSL)]
            + [pltpu.VMEM((RR, 128), jnp.float32)
               for _ in range(3)]
            + [pltpu.SemaphoreType.DMA for _ in range(NF)]
            + [pltpu.SemaphoreType.DMA for _ in range(NCSL)]
        ),
    )
    def sc_k(f_hbm, l_hbm, c_hbm, ff_hbm, fc_hbm, cc_hbm, idx_v, *rest):
        fbufs = rest[:NF]
        cbufs = rest[NF:NF + NCSL]
        ffr, fcr, ccr = rest[NF + NCSL:NF + NCSL + 3]
        fsems = rest[NF + NCSL + 3:NF + NCSL + 3 + NF]
        csems = rest[NF + NCSL + 3 + NF:]
        wid = lax.axis_index("s") * NC + lax.axis_index("c")
        base = wid * bw
        pltpu.sync_copy(l_hbm.at[pl.ds(base, bw)], idx_v)

        def fcopy(ch, s):
            return pltpu.make_async_copy(
                f_hbm.at[pl.ds(base + ch * CH, CH)], fbufs[s], fsems[s])

        def ccopy(ch, s):
            return pltpu.make_async_copy(
                c_hbm.at[idx_v.at[pl.ds(ch * CH, CH)]], cbufs[s], csems[s])

        def compute(ch, fb, cb):
            def row_body(r, _2):
                def inner(k, carry):
                    a, b, c = carry
                    off = pl.multiple_of(k * L, L)
                    fv = fb[r, pl.ds(off, L)]
                    cv = cb[r, pl.ds(off, L)]
                    return (a + fv * fv, b + fv * cv, c + cv * cv)

                z = jnp.zeros((L,), jnp.float32)
                a, b, c = lax.fori_loop(0, KV, inner, (z, z, z), unroll=8)
                flat = (ch * CH + r) * L
                rrow = flat // 128
                roff = pl.multiple_of(flat % 128, L)
                ffr[rrow, pl.ds(roff, L)] = a
                fcr[rrow, pl.ds(roff, L)] = b
                ccr[rrow, pl.ds(roff, L)] = c
                return 0

            lax.fori_loop(0, CH, row_body, 0)

        GRP = 8
        NG = NCH // GRP
        for s in range(NF):
            fcopy(s, s).start()
        for s in range(NCSL):
            ccopy(s, s).start()

        def group_body(g, _):
            c0 = g * GRP
            for s in range(GRP):
                ch = c0 + s
                fs = s % NF
                cs = s % NCSL
                fcopy(ch, fs).wait()
                ccopy(ch, cs).wait()
                compute(ch, fbufs[fs], cbufs[cs])

                @pl.when(ch + NF < NCH)
                def _(ch=ch, fs=fs):
                    fcopy(ch + NF, fs).start()

                @pl.when(ch + NCSL < NCH)
                def _(ch=ch, cs=cs):
                    ccopy(ch + NCSL, cs).start()

            return 0

        lax.fori_loop(0, NG, group_body, 0)
        rbase = wid * RR
        pltpu.sync_copy(ffr, ff_hbm.at[pl.ds(rbase, RR)])
        pltpu.sync_copy(fcr, fc_hbm.at[pl.ds(rbase, RR)])
        pltpu.sync_copy(ccr, cc_hbm.at[pl.ds(rbase, RR)])

    return sc_k(features, labels, centers)


def _tc_epilogue(ff, fc, cc, B):
    N = ff.shape[0]
    G = 128 // L
    BLK = 1024
    NBLK = N // BLK

    def body(ff_ref, fc_ref, cc_ref, out_ref, acc_ref):
        i = pl.program_id(0)

        @pl.when(i == 0)
        def _():
            acc_ref[0, 0] = 0.0

        lane = lax.broadcasted_iota(jnp.int32, (128, G), 0)
        grp = lax.broadcasted_iota(jnp.int32, (128, G), 1)
        sel = (lane // L == grp).astype(jnp.float32)
        ffs = jax.lax.dot(ff_ref[...], sel,
                          preferred_element_type=jnp.float32)
        fcs = jax.lax.dot(fc_ref[...], sel,
                          preferred_element_type=jnp.float32)
        ccs = jax.lax.dot(cc_ref[...], sel,
                          preferred_element_type=jnp.float32)
        d = jnp.maximum(jnp.sqrt(ffs), 1e-12)
        li = ffs / (d * d) + ccs - 2.0 * fcs / d
        acc_ref[0, 0] += jnp.sum(li)

        @pl.when(i == NBLK - 1)
        def _():
            out_ref[...] = jnp.full((1, 1), acc_ref[0, 0] / (2.0 * B),
                                    dtype=jnp.float32)

    out = pl.pallas_call(
        body,
        grid=(NBLK,),
        in_specs=[pl.BlockSpec((BLK, 128), lambda i: (i, 0))] * 3,
        out_specs=pl.BlockSpec((1, 1), lambda i: (0, 0)),
        out_shape=jax.ShapeDtypeStruct((1, 1), jnp.float32),
        scratch_shapes=[pltpu.SMEM((1, 1), jnp.float32)],
    )(ff, fc, cc)
    return out[0, 0]


def kernel(features, labels, centers):
    B = features.shape[0]
    labels32 = labels.astype(jnp.int32)
    ff, fc, cc = _sc_partials(features, labels32, centers)
    return _tc_epilogue(ff, fc, cc, B)


# speedup vs baseline: 1.2530x; 1.2530x over previous
import functools

import jax
import jax.numpy as jnp
from jax import lax
from jax.experimental import pallas as pl
from jax.experimental.pallas import tpu as pltpu
from jax.experimental.pallas import tpu_sc as plsc

L = 16
NC = 2
NS = 16
NW = NC * NS


def _sc_partials(features, labels, centers):
    B, D = features.shape
    bw = B // NW
    CH = 8
    NF = 4
    NCSL = 4
    NCH = bw // CH
    KV = D // L
    RR = bw * L // 128

    mesh = plsc.VectorSubcoreMesh(core_axis_name="c", subcore_axis_name="s")

    @functools.partial(
        pl.kernel,
        mesh=mesh,
        compiler_params=pltpu.CompilerParams(use_tc_tiling_on_sc=True),
        out_type=[jax.ShapeDtypeStruct((B * L // 128, 128), jnp.float32)
                  for _ in range(3)],
        scratch_types=(
            [pltpu.VMEM((bw,), jnp.int32)]
            + [pltpu.VMEM((CH, D), jnp.float32)
               for _ in range(NF)]
            + [pltpu.VMEM((CH, D), jnp.float32)
               for _ in range(NCSL)]
            + [pltpu.VMEM((RR, 128), jnp.float32)
               for _ in range(3)]
            + [pltpu.SemaphoreType.DMA for _ in range(NF)]
            + [pltpu.SemaphoreType.DMA for _ in range(NCSL)]
        ),
    )
    def sc_k(f_hbm, l_hbm, c_hbm, ff_hbm, fc_hbm, cc_hbm, idx_v, *rest):
        fbufs = rest[:NF]
        cbufs = rest[NF:NF + NCSL]
        ffr, fcr, ccr = rest[NF + NCSL:NF + NCSL + 3]
        fsems = rest[NF + NCSL + 3:NF + NCSL + 3 + NF]
        csems = rest[NF + NCSL + 3 + NF:]
        wid = lax.axis_index("s") * NC + lax.axis_index("c")
        base = wid * bw
        pltpu.sync_copy(l_hbm.at[pl.ds(base, bw)], idx_v)

        def fcopy(ch, s):
            return pltpu.make_async_copy(
                f_hbm.at[pl.ds(base + ch * CH, CH)], fbufs[s], fsems[s])

        def ccopy(ch, s):
            return pltpu.make_async_copy(
                c_hbm.at[idx_v.at[pl.ds(ch * CH, CH)]], cbufs[s], csems[s])

        def compute(ch, fb, cb):
            def row_body(r, _2):
                def inner(k, carry):
                    a, b, c = carry
                    off = pl.multiple_of(k * L, L)
                    fv = fb[r, pl.ds(off, L)]
                    cv = cb[r, pl.ds(off, L)]
                    return (a + fv * fv, b + fv * cv, c + cv * cv)

                z = jnp.zeros((L,), jnp.float32)
                a, b, c = lax.fori_loop(0, KV, inner, (z, z, z), unroll=8)
                flat = (ch * CH + r) * L
                rrow = flat // 128
                roff = pl.multiple_of(flat % 128, L)
                ffr[rrow, pl.ds(roff, L)] = a
                fcr[rrow, pl.ds(roff, L)] = b
                ccr[rrow, pl.ds(roff, L)] = c
                return 0

            lax.fori_loop(0, CH, row_body, 0)

        GRP = 4
        NG = NCH // GRP
        for s in range(NF):
            fcopy(s, s).start()
        for s in range(NCSL):
            ccopy(s, s).start()

        def group_body(g, _):
            c0 = g * GRP
            for s in range(GRP):
                ch = c0 + s
                fs = s % NF
                cs = s % NCSL
                fcopy(ch, fs).wait()
                ccopy(ch, cs).wait()
                compute(ch, fbufs[fs], cbufs[cs])

                @pl.when(ch + NF < NCH)
                def _(ch=ch, fs=fs):
                    fcopy(ch + NF, fs).start()

                @pl.when(ch + NCSL < NCH)
                def _(ch=ch, cs=cs):
                    ccopy(ch + NCSL, cs).start()

            return 0

        lax.fori_loop(0, NG, group_body, 0)
        rbase = wid * RR
        pltpu.sync_copy(ffr, ff_hbm.at[pl.ds(rbase, RR)])
        pltpu.sync_copy(fcr, fc_hbm.at[pl.ds(rbase, RR)])
        pltpu.sync_copy(ccr, cc_hbm.at[pl.ds(rbase, RR)])

    return sc_k(features, labels, centers)


def _tc_epilogue(ff, fc, cc, B):
    N = ff.shape[0]
    G = 128 // L
    BLK = 1024
    NBLK = N // BLK

    def body(ff_ref, fc_ref, cc_ref, out_ref, acc_ref):
        i = pl.program_id(0)

        @pl.when(i == 0)
        def _():
            acc_ref[0, 0] = 0.0

        lane = lax.broadcasted_iota(jnp.int32, (128, G), 0)
        grp = lax.broadcasted_iota(jnp.int32, (128, G), 1)
        sel = (lane // L == grp).astype(jnp.float32)
        ffs = jax.lax.dot(ff_ref[...], sel,
                          preferred_element_type=jnp.float32)
        fcs = jax.lax.dot(fc_ref[...], sel,
                          preferred_element_type=jnp.float32)
        ccs = jax.lax.dot(cc_ref[...], sel,
                          preferred_element_type=jnp.float32)
        d = jnp.maximum(jnp.sqrt(ffs), 1e-12)
        li = ffs / (d * d) + ccs - 2.0 * fcs / d
        acc_ref[0, 0] += jnp.sum(li)

        @pl.when(i == NBLK - 1)
        def _():
            out_ref[...] = jnp.full((1, 1), acc_ref[0, 0] / (2.0 * B),
                                    dtype=jnp.float32)

    out = pl.pallas_call(
        body,
        grid=(NBLK,),
        in_specs=[pl.BlockSpec((BLK, 128), lambda i: (i, 0))] * 3,
        out_specs=pl.BlockSpec((1, 1), lambda i: (0, 0)),
        out_shape=jax.ShapeDtypeStruct((1, 1), jnp.float32),
        scratch_shapes=[pltpu.SMEM((1, 1), jnp.float32)],
    )(ff, fc, cc)
    return out[0, 0]


def kernel(features, labels, centers):
    B = features.shape[0]
    labels32 = labels.astype(jnp.int32)
    ff, fc, cc = _sc_partials(features, labels32, centers)
    return _tc_epilogue(ff, fc, cc, B)
